# baseline (speedup 1.0000x reference)
import functools

import jax
import jax.numpy as jnp
from jax import lax
from jax.experimental import pallas as pl
from jax.experimental.pallas import tpu as pltpu
from jax.experimental.pallas import tpu_sc as plsc

N_ENT = 1000000
N_REL = 1000
DIM = 64
GAMMA = 12.0
B = 16384

NC = 2
NS = 16
L = 16
NW = NC * NS


def _rsqrt_nr(x):
    i = plsc.bitcast(x, jnp.int32)
    i = jnp.int32(0x5F3759DF) - (i >> 1)
    y = plsc.bitcast(i, jnp.float32)
    for _ in range(3):
        y = y * (1.5 - 0.5 * x * y * y)
    return y


def _build(n_ent, n_rel, dim, b, gamma, interpret=False):
    tpw = b // NW
    pair = 16
    qrows = 4
    n_pairs = tpw // pair
    n_bodies = n_pairs // 2
    jc_n = dim // L
    row_bytes_rows = (3 * pair, dim)

    mesh = plsc.VectorSubcoreMesh(core_axis_name="c", subcore_axis_name="s")

    @functools.partial(
        pl.kernel,
        out_type=jax.ShapeDtypeStruct((b,), jnp.float32),
        mesh=mesh,
        interpret=interpret,
        compiler_params=pltpu.CompilerParams(needs_layout_passes=False),
        scratch_types=[
            pltpu.VMEM((tpw,), jnp.int32),
            pltpu.VMEM((tpw,), jnp.int32),
            pltpu.VMEM((tpw,), jnp.int32),
            pltpu.VMEM((2 * pair,), jnp.int32),
            pltpu.VMEM((2 * pair,), jnp.int32),
            pltpu.VMEM((pair, dim), jnp.float32),
            pltpu.VMEM((pair, dim), jnp.float32),
            pltpu.VMEM((pair, dim), jnp.float32),
            pltpu.VMEM((pair, dim), jnp.float32),
            pltpu.VMEM((pair, dim), jnp.float32),
            pltpu.VMEM((pair, dim), jnp.float32),
            pltpu.VMEM((qrows, dim * dim), jnp.float32),
            pltpu.VMEM((qrows, dim * dim), jnp.float32),
            pltpu.VMEM((qrows, dim * dim), jnp.float32),
            pltpu.VMEM((qrows, dim * dim), jnp.float32),
            pltpu.VMEM((6 * pair, L), jnp.float32),
            pltpu.VMEM((6 * pair, L), jnp.float32),
            pltpu.VMEM((tpw,), jnp.float32),
            pltpu.VMEM(row_bytes_rows, jnp.float32),
            pltpu.SemaphoreType.DMA,
            pltpu.SemaphoreType.DMA,
            pltpu.SemaphoreType.DMA,
            pltpu.SemaphoreType.DMA,
            pltpu.SemaphoreType.DMA,
            pltpu.SemaphoreType.DMA,
            pltpu.SemaphoreType.DMA,
        ],
    )
    def trans_r(ent_hbm, rel_hbm, mat_hbm, h_hbm, r_hbm, t_hbm, out_hbm,
                h_vm, t_vm, r_vm, ridxv0, ridxv1, he0, te0, rv0, he1, te1,
                rv1, m0, m1, m2, m3, red0, red1, score_all, drain_rows,
                sem_i, semR0, semR1, semM0, semM1, semM2, semM3):
        wid = lax.axis_index("s") * NC + lax.axis_index("c")
        base_w = wid * tpw
        mslots = [m0, m1, m2, m3]
        msems = [semM0, semM1, semM2, semM3]

        def pair_idx_sign(p):
            goff = jnp.minimum(p * pair, tpw - pair)
            rr = r_vm[pl.ds(goff, pair)]
            return lax.rem(rr, n_rel), jnp.where(
                rr < n_rel, 1.0, -1.0).astype(jnp.float32)

        def issue_rows(p, he_b, te_b, rv_b, ridx, semR):
            goff = p * pair
            hh = h_vm[pl.ds(goff, pair)]
            tt = t_vm[pl.ds(goff, pair)]
            for c in range(pair):
                pltpu.async_copy(ent_hbm.at[hh[c]], he_b.at[c], semR)
                pltpu.async_copy(ent_hbm.at[tt[c]], te_b.at[c], semR)
                pltpu.async_copy(rel_hbm.at[ridx[c]], rv_b.at[c], semR)

        def wait_rows(semR):
            pltpu.make_async_copy(
                ent_hbm.at[pl.ds(0, 3 * pair)], drain_rows, semR).wait()

        def store_quarter_idx(ridxv_ref, ridx):
            lane = lax.iota(jnp.int32, L)
            pos = 8 * (lane >> 2) + (lane & 3)
            plsc.store_scatter(ridxv_ref, [pos], ridx)

        def issue_quarter(ridxv_ref, q_in_pair, slot):
            pltpu.async_copy(
                mat_hbm.at[ridxv_ref.at[pl.ds(q_in_pair * 8, qrows)]],
                mslots[slot], msems[slot])

        def wait_quarter(slot):
            pltpu.make_async_copy(
                mat_hbm.at[pl.ds(0, qrows)], mslots[slot], msems[slot]).wait()

        def triple_body(tri, mat_ref, mi, he_b, te_b, rv_b, red_b, carry):
            hrow = [he_b[tri, pl.ds(c * L, L)] for c in range(jc_n)]
            trow = [te_b[tri, pl.ds(c * L, L)] for c in range(jc_n)]
            acc_h = [jnp.zeros((L,), jnp.float32) for _ in range(jc_n)]
            acc_t = [jnp.zeros((L,), jnp.float32) for _ in range(jc_n)]
            for k in range(dim):
                hk = hrow[k // L][k % L]
                tk = trow[k // L][k % L]
                for jc in range(jc_n):
                    m = mat_ref[mi, pl.ds(k * dim + jc * L, L)]
                    acc_h[jc] = acc_h[jc] + hk * m
                    acc_t[jc] = acc_t[jc] + tk * m
            rvc = [rv_b[tri, pl.ds(jc * L, L)] for jc in range(jc_n)]
            grams = [jnp.zeros((L,), jnp.float32) for _ in range(6)]
            for jc in range(jc_n):
                ah, at, rv = acc_h[jc], acc_t[jc], rvc[jc]
                grams[0] = grams[0] + ah * ah
                grams[1] = grams[1] + at * at
                grams[2] = grams[2] + rv * rv
                grams[3] = grams[3] + ah * rv
                grams[4] = grams[4] + ah * at
                grams[5] = grams[5] + rv * at
            for q in range(6):
                red_b[q * pair + tri, :] = grams[q]
            return carry

        def epilogue(p, red_b, sign_vec):
            lane = lax.iota(jnp.int32, L)
            red = []
            for q in range(6):
                acc = jnp.zeros((L,), jnp.float32)
                for col in range(L):
                    acc = acc + plsc.load_gather(
                        red_b, [jnp.int32(q * pair) + lane,
                                jnp.full((L,), col, jnp.int32)])
                red.append(acc)
            sh, st, sr, phr, pht, prt = red

            def inv_norm(ssq):
                n = ssq * _rsqrt_nr(jnp.maximum(ssq, 1e-30))
                return 1.0 / jnp.maximum(n, 1e-12)

            ih, it_ = inv_norm(sh), inv_norm(st)
            irs = sign_vec * inv_norm(sr)
            nsq = (ih * ih * sh + it_ * it_ * st + irs * irs * sr
                   + 2.0 * (ih * irs * phr - ih * it_ * pht - irs * it_ * prt))
            nsq = jnp.maximum(nsq, 0.0)
            dist = nsq * _rsqrt_nr(jnp.maximum(nsq, 1e-30))
            score_all[pl.ds(p * pair, pair)] = gamma - dist

        def quarter_compute(j, ridxv_ref_next, he_b, te_b, rv_b, red_b,
                            prefetch_ok):
            slot = j % 4
            pre_slot = (j + 2) % 4
            ridx_ref, q_in_pair, guarded = ridxv_ref_next
            if guarded:
                @pl.when(prefetch_ok)
                def _():
                    issue_quarter(ridx_ref, q_in_pair, pre_slot)
            else:
                issue_quarter(ridx_ref, q_in_pair, pre_slot)
            wait_quarter(slot)

            @plsc.parallel_loop((j % 4) * qrows, (j % 4) * qrows + qrows,
                                unroll=2)
            def _(t):
                triple_body(t, mslots[slot], t - (j % 4) * qrows,
                            he_b, te_b, rv_b, red_b, 0)

        def body(k, sign0):
            p0 = 2 * k
            p1 = 2 * k + 1
            not_last = k < n_bodies - 1
            ridx1, sign1 = pair_idx_sign(p1)
            store_quarter_idx(ridxv1, ridx1)
            issue_rows(p1, he1, te1, rv1, ridx1, semR1)
            wait_rows(semR0)
            quarter_compute(0, (ridxv0, 2, False), he0, te0, rv0, red0, True)
            quarter_compute(1, (ridxv0, 3, False), he0, te0, rv0, red0, True)
            quarter_compute(2, (ridxv1, 0, False), he0, te0, rv0, red0, True)
            quarter_compute(3, (ridxv1, 1, False), he0, te0, rv0, red0, True)
            epilogue(p0, red0, sign0)
            ridx2, sign2 = pair_idx_sign(p0 + 2)
            @pl.when(not_last)
            def _():
                store_quarter_idx(ridxv0, ridx2)
                issue_rows(p0 + 2, he0, te0, rv0, ridx2, semR0)
            wait_rows(semR1)
            quarter_compute(4, (ridxv1, 2, False), he1, te1, rv1, red1, True)
            quarter_compute(5, (ridxv1, 3, False), he1, te1, rv1, red1, True)
            quarter_compute(6, (ridxv0, 0, True), he1, te1, rv1, red1, not_last)
            quarter_compute(7, (ridxv0, 1, True), he1, te1, rv1, red1, not_last)
            epilogue(p1, red1, sign1)
            return sign2

        cp_h = pltpu.async_copy(h_hbm.at[pl.ds(base_w, tpw)], h_vm, sem_i)
        cp_t = pltpu.async_copy(t_hbm.at[pl.ds(base_w, tpw)], t_vm, sem_i)
        cp_r = pltpu.async_copy(r_hbm.at[pl.ds(base_w, tpw)], r_vm, sem_i)
        cp_h.wait()
        cp_t.wait()
        cp_r.wait()
        ridx0, sign0 = pair_idx_sign(0)
        store_quarter_idx(ridxv0, ridx0)
        issue_rows(0, he0, te0, rv0, ridx0, semR0)
        issue_quarter(ridxv0, 0, 0)
        issue_quarter(ridxv0, 1, 1)
        lax.fori_loop(0, n_bodies, body, sign0)
        pltpu.sync_copy(score_all, out_hbm.at[pl.ds(base_w, tpw)])

    return trans_r


_trans_r_full = None


def kernel(ent_embed, rel_embed, mat_embed, h, r, t):
    global _trans_r_full
    if _trans_r_full is None:
        _trans_r_full = _build(N_ENT, N_REL, DIM, B, GAMMA)
    return _trans_r_full(ent_embed, rel_embed, mat_embed, h, r, t)


# speedup vs baseline: 1.1862x; 1.1862x over previous
import functools

import jax
import jax.numpy as jnp
from jax import lax
from jax.experimental import pallas as pl
from jax.experimental.pallas import tpu as pltpu
from jax.experimental.pallas import tpu_sc as plsc

N_ENT = 1000000
N_REL = 1000
DIM = 64
GAMMA = 12.0
B = 16384

NC = 2
NS = 16
L = 16
NW = NC * NS


def _rsqrt_nr(x):
    i = plsc.bitcast(x, jnp.int32)
    i = jnp.int32(0x5F3759DF) - (i >> 1)
    y = plsc.bitcast(i, jnp.float32)
    for _ in range(3):
        y = y * (1.5 - 0.5 * x * y * y)
    return y


def _build(n_ent, n_rel, dim, b, gamma, interpret=False):
    tpw = b // NW
    pair = 16
    qrows = 4
    n_pairs = tpw // pair
    n_bodies = n_pairs // 2
    jc_n = dim // L
    row_bytes_rows = (3 * pair, dim)

    mesh = plsc.VectorSubcoreMesh(core_axis_name="c", subcore_axis_name="s")

    @functools.partial(
        pl.kernel,
        out_type=jax.ShapeDtypeStruct((b,), jnp.float32),
        mesh=mesh,
        interpret=interpret,
        compiler_params=pltpu.CompilerParams(needs_layout_passes=False),
        scratch_types=[
            pltpu.VMEM((tpw,), jnp.int32),
            pltpu.VMEM((tpw,), jnp.int32),
            pltpu.VMEM((tpw,), jnp.int32),
            pltpu.VMEM((2 * pair,), jnp.int32),
            pltpu.VMEM((2 * pair,), jnp.int32),
            pltpu.VMEM((pair, dim), jnp.float32),
            pltpu.VMEM((pair, dim), jnp.float32),
            pltpu.VMEM((pair, dim), jnp.float32),
            pltpu.VMEM((pair, dim), jnp.float32),
            pltpu.VMEM((pair, dim), jnp.float32),
            pltpu.VMEM((pair, dim), jnp.float32),
            pltpu.VMEM((qrows, dim * dim), jnp.float32),
            pltpu.VMEM((qrows, dim * dim), jnp.float32),
            pltpu.VMEM((qrows, dim * dim), jnp.float32),
            pltpu.VMEM((qrows, dim * dim), jnp.float32),
            pltpu.VMEM((6 * pair, L), jnp.float32),
            pltpu.VMEM((6 * pair, L), jnp.float32),
            pltpu.VMEM((tpw,), jnp.float32),
            pltpu.VMEM(row_bytes_rows, jnp.float32),
            pltpu.SemaphoreType.DMA,
            pltpu.SemaphoreType.DMA,
            pltpu.SemaphoreType.DMA,
            pltpu.SemaphoreType.DMA,
            pltpu.SemaphoreType.DMA,
            pltpu.SemaphoreType.DMA,
            pltpu.SemaphoreType.DMA,
        ],
    )
    def trans_r(ent_hbm, rel_hbm, mat_hbm, h_hbm, r_hbm, t_hbm, out_hbm,
                h_vm, t_vm, r_vm, ridxv0, ridxv1, he0, te0, rv0, he1, te1,
                rv1, m0, m1, m2, m3, red0, red1, score_all, drain_rows,
                sem_i, semR0, semR1, semM0, semM1, semM2, semM3):
        wid = lax.axis_index("s") * NC + lax.axis_index("c")
        base_w = wid * tpw
        mslots = [m0, m1, m2, m3]
        msems = [semM0, semM1, semM2, semM3]

        def pair_idx_sign(p):
            goff = jnp.minimum(p * pair, tpw - pair)
            rr = r_vm[pl.ds(goff, pair)]
            return lax.rem(rr, n_rel), jnp.where(
                rr < n_rel, 1.0, -1.0).astype(jnp.float32)

        def issue_rows(p, he_b, te_b, rv_b, ridx, semR):
            goff = p * pair
            hh = h_vm[pl.ds(goff, pair)]
            tt = t_vm[pl.ds(goff, pair)]
            for c in range(pair):
                pltpu.async_copy(ent_hbm.at[hh[c]], he_b.at[c], semR)
                pltpu.async_copy(ent_hbm.at[tt[c]], te_b.at[c], semR)
                pltpu.async_copy(rel_hbm.at[ridx[c]], rv_b.at[c], semR)

        def wait_rows(semR):
            pltpu.make_async_copy(
                ent_hbm.at[pl.ds(0, 3 * pair)], drain_rows, semR).wait()

        def store_quarter_idx(ridxv_ref, ridx):
            lane = lax.iota(jnp.int32, L)
            pos = 8 * (lane >> 2) + (lane & 3)
            plsc.store_scatter(ridxv_ref, [pos], ridx)

        def issue_quarter(ridxv_ref, q_in_pair, slot):
            pltpu.async_copy(
                mat_hbm.at[ridxv_ref.at[pl.ds(q_in_pair * 8, qrows)]],
                mslots[slot], msems[slot])

        def wait_quarter(slot):
            pltpu.make_async_copy(
                mat_hbm.at[pl.ds(0, qrows)], mslots[slot], msems[slot]).wait()

        def triple_body(tri, mat_ref, mi, he_b, te_b, rv_b, red_b, carry):
            hrow = [he_b[tri, pl.ds(c * L, L)] for c in range(jc_n)]
            trow = [te_b[tri, pl.ds(c * L, L)] for c in range(jc_n)]
            acc_h = [jnp.zeros((L,), jnp.float32) for _ in range(jc_n)]
            acc_t = [jnp.zeros((L,), jnp.float32) for _ in range(jc_n)]
            for k in range(dim):
                hk = hrow[k // L][k % L]
                tk = trow[k // L][k % L]
                for jc in range(jc_n):
                    m = mat_ref[mi, pl.ds(k * dim + jc * L, L)]
                    acc_h[jc] = acc_h[jc] + hk * m
                    acc_t[jc] = acc_t[jc] + tk * m
            rvc = [rv_b[tri, pl.ds(jc * L, L)] for jc in range(jc_n)]
            grams = [jnp.zeros((L,), jnp.float32) for _ in range(6)]
            for jc in range(jc_n):
                ah, at, rv = acc_h[jc], acc_t[jc], rvc[jc]
                grams[0] = grams[0] + ah * ah
                grams[1] = grams[1] + at * at
                grams[2] = grams[2] + rv * rv
                grams[3] = grams[3] + ah * rv
                grams[4] = grams[4] + ah * at
                grams[5] = grams[5] + rv * at
            for q in range(6):
                red_b[q * pair + tri, :] = grams[q]
            return carry

        def epilogue(p, red_b, sign_vec):
            lane = lax.iota(jnp.int32, L)
            red = []
            for q in range(6):
                acc = jnp.zeros((L,), jnp.float32)
                for col in range(L):
                    acc = acc + plsc.load_gather(
                        red_b, [jnp.int32(q * pair) + lane,
                                jnp.full((L,), col, jnp.int32)])
                red.append(acc)
            sh, st, sr, phr, pht, prt = red

            def inv_norm(ssq):
                n = ssq * _rsqrt_nr(jnp.maximum(ssq, 1e-30))
                return 1.0 / jnp.maximum(n, 1e-12)

            ih, it_ = inv_norm(sh), inv_norm(st)
            irs = sign_vec * inv_norm(sr)
            nsq = (ih * ih * sh + it_ * it_ * st + irs * irs * sr
                   + 2.0 * (ih * irs * phr - ih * it_ * pht - irs * it_ * prt))
            nsq = jnp.maximum(nsq, 0.0)
            dist = nsq * _rsqrt_nr(jnp.maximum(nsq, 1e-30))
            score_all[pl.ds(p * pair, pair)] = gamma - dist

        def quarter_compute(j, ridxv_ref_next, he_b, te_b, rv_b, red_b,
                            prefetch_ok):
            slot = j % 4
            pre_slot = (j + 2) % 4
            ridx_ref, q_in_pair, guarded = ridxv_ref_next
            if guarded:
                @pl.when(prefetch_ok)
                def _():
                    issue_quarter(ridx_ref, q_in_pair, pre_slot)
            else:
                issue_quarter(ridx_ref, q_in_pair, pre_slot)
            wait_quarter(slot)

            @plsc.parallel_loop((j % 4) * qrows, (j % 4) * qrows + qrows)
            def _(t):
                triple_body(t, mslots[slot], t - (j % 4) * qrows,
                            he_b, te_b, rv_b, red_b, 0)

        def body(k, sign0):
            p0 = 2 * k
            p1 = 2 * k + 1
            not_last = k < n_bodies - 1
            ridx1, sign1 = pair_idx_sign(p1)
            store_quarter_idx(ridxv1, ridx1)
            issue_rows(p1, he1, te1, rv1, ridx1, semR1)
            wait_rows(semR0)
            quarter_compute(0, (ridxv0, 2, False), he0, te0, rv0, red0, True)
            quarter_compute(1, (ridxv0, 3, False), he0, te0, rv0, red0, True)
            quarter_compute(2, (ridxv1, 0, False), he0, te0, rv0, red0, True)
            quarter_compute(3, (ridxv1, 1, False), he0, te0, rv0, red0, True)
            epilogue(p0, red0, sign0)
            ridx2, sign2 = pair_idx_sign(p0 + 2)
            @pl.when(not_last)
            def _():
                store_quarter_idx(ridxv0, ridx2)
                issue_rows(p0 + 2, he0, te0, rv0, ridx2, semR0)
            wait_rows(semR1)
            quarter_compute(4, (ridxv1, 2, False), he1, te1, rv1, red1, True)
            quarter_compute(5, (ridxv1, 3, False), he1, te1, rv1, red1, True)
            quarter_compute(6, (ridxv0, 0, True), he1, te1, rv1, red1, not_last)
            quarter_compute(7, (ridxv0, 1, True), he1, te1, rv1, red1, not_last)
            epilogue(p1, red1, sign1)
            return sign2

        cp_h = pltpu.async_copy(h_hbm.at[pl.ds(base_w, tpw)], h_vm, sem_i)
        cp_t = pltpu.async_copy(t_hbm.at[pl.ds(base_w, tpw)], t_vm, sem_i)
        cp_r = pltpu.async_copy(r_hbm.at[pl.ds(base_w, tpw)], r_vm, sem_i)
        cp_h.wait()
        cp_t.wait()
        cp_r.wait()
        ridx0, sign0 = pair_idx_sign(0)
        store_quarter_idx(ridxv0, ridx0)
        issue_rows(0, he0, te0, rv0, ridx0, semR0)
        issue_quarter(ridxv0, 0, 0)
        issue_quarter(ridxv0, 1, 1)
        lax.fori_loop(0, n_bodies, body, sign0)
        pltpu.sync_copy(score_all, out_hbm.at[pl.ds(base_w, tpw)])

    return trans_r


_trans_r_full = None


def kernel(ent_embed, rel_embed, mat_embed, h, r, t):
    global _trans_r_full
    if _trans_r_full is None:
        _trans_r_full = _build(N_ENT, N_REL, DIM, B, GAMMA)
    return _trans_r_full(ent_embed, rel_embed, mat_embed, h, r, t)


# speedup vs baseline: 1.1871x; 1.0008x over previous
import functools

import jax
import jax.numpy as jnp
from jax import lax
from jax.experimental import pallas as pl
from jax.experimental.pallas import tpu as pltpu
from jax.experimental.pallas import tpu_sc as plsc

N_ENT = 1000000
N_REL = 1000
DIM = 64
GAMMA = 12.0
B = 16384

NC = 2
NS = 16
L = 16
NW = NC * NS


def _rsqrt_nr(x):
    i = plsc.bitcast(x, jnp.int32)
    i = jnp.int32(0x5F3759DF) - (i >> 1)
    y = plsc.bitcast(i, jnp.float32)
    for _ in range(3):
        y = y * (1.5 - 0.5 * x * y * y)
    return y


def _build(n_ent, n_rel, dim, b, gamma, interpret=False):
    tpw = b // NW
    pair = 16
    qrows = 4
    n_pairs = tpw // pair
    n_bodies = n_pairs // 2
    jc_n = dim // L
    row_bytes_rows = (3 * pair, dim)

    mesh = plsc.VectorSubcoreMesh(core_axis_name="c", subcore_axis_name="s")

    @functools.partial(
        pl.kernel,
        out_type=jax.ShapeDtypeStruct((b,), jnp.float32),
        mesh=mesh,
        interpret=interpret,
        compiler_params=pltpu.CompilerParams(
            needs_layout_passes=False,
            disable_bounds_checks=True,
            disable_semaphore_checks=True),
        scratch_types=[
            pltpu.VMEM((tpw,), jnp.int32),
            pltpu.VMEM((tpw,), jnp.int32),
            pltpu.VMEM((tpw,), jnp.int32),
            pltpu.VMEM((2 * pair,), jnp.int32),
            pltpu.VMEM((2 * pair,), jnp.int32),
            pltpu.VMEM((pair, dim), jnp.float32),
            pltpu.VMEM((pair, dim), jnp.float32),
            pltpu.VMEM((pair, dim), jnp.float32),
            pltpu.VMEM((pair, dim), jnp.float32),
            pltpu.VMEM((pair, dim), jnp.float32),
            pltpu.VMEM((pair, dim), jnp.float32),
            pltpu.VMEM((qrows, dim * dim), jnp.float32),
            pltpu.VMEM((qrows, dim * dim), jnp.float32),
            pltpu.VMEM((qrows, dim * dim), jnp.float32),
            pltpu.VMEM((qrows, dim * dim), jnp.float32),
            pltpu.VMEM((6 * pair, L), jnp.float32),
            pltpu.VMEM((6 * pair, L), jnp.float32),
            pltpu.VMEM((tpw,), jnp.float32),
            pltpu.VMEM(row_bytes_rows, jnp.float32),
            pltpu.SemaphoreType.DMA,
            pltpu.SemaphoreType.DMA,
            pltpu.SemaphoreType.DMA,
            pltpu.SemaphoreType.DMA,
            pltpu.SemaphoreType.DMA,
            pltpu.SemaphoreType.DMA,
            pltpu.SemaphoreType.DMA,
        ],
    )
    def trans_r(ent_hbm, rel_hbm, mat_hbm, h_hbm, r_hbm, t_hbm, out_hbm,
                h_vm, t_vm, r_vm, ridxv0, ridxv1, he0, te0, rv0, he1, te1,
                rv1, m0, m1, m2, m3, red0, red1, score_all, drain_rows,
                sem_i, semR0, semR1, semM0, semM1, semM2, semM3):
        wid = lax.axis_index("s") * NC + lax.axis_index("c")
        base_w = wid * tpw
        mslots = [m0, m1, m2, m3]
        msems = [semM0, semM1, semM2, semM3]

        def pair_idx_sign(p):
            goff = jnp.minimum(p * pair, tpw - pair)
            rr = r_vm[pl.ds(goff, pair)]
            return lax.rem(rr, n_rel), jnp.where(
                rr < n_rel, 1.0, -1.0).astype(jnp.float32)

        def issue_rows(p, he_b, te_b, rv_b, ridx, semR):
            goff = p * pair
            hh = h_vm[pl.ds(goff, pair)]
            tt = t_vm[pl.ds(goff, pair)]
            for c in range(pair):
                pltpu.async_copy(ent_hbm.at[hh[c]], he_b.at[c], semR)
                pltpu.async_copy(ent_hbm.at[tt[c]], te_b.at[c], semR)
                pltpu.async_copy(rel_hbm.at[ridx[c]], rv_b.at[c], semR)

        def wait_rows(semR):
            pltpu.make_async_copy(
                ent_hbm.at[pl.ds(0, 3 * pair)], drain_rows, semR).wait()

        def store_quarter_idx(ridxv_ref, ridx):
            lane = lax.iota(jnp.int32, L)
            pos = 8 * (lane >> 2) + (lane & 3)
            plsc.store_scatter(ridxv_ref, [pos], ridx)

        def issue_quarter(ridxv_ref, q_in_pair, slot):
            pltpu.async_copy(
                mat_hbm.at[ridxv_ref.at[pl.ds(q_in_pair * 8, qrows)]],
                mslots[slot], msems[slot])

        def wait_quarter(slot):
            pltpu.make_async_copy(
                mat_hbm.at[pl.ds(0, qrows)], mslots[slot], msems[slot]).wait()

        def triple_body(tri, mat_ref, mi, he_b, te_b, rv_b, red_b, carry):
            hrow = [he_b[tri, pl.ds(c * L, L)] for c in range(jc_n)]
            trow = [te_b[tri, pl.ds(c * L, L)] for c in range(jc_n)]
            acc_h = [jnp.zeros((L,), jnp.float32) for _ in range(jc_n)]
            acc_t = [jnp.zeros((L,), jnp.float32) for _ in range(jc_n)]
            for k in range(dim):
                hk = hrow[k // L][k % L]
                tk = trow[k // L][k % L]
                for jc in range(jc_n):
                    m = mat_ref[mi, pl.ds(k * dim + jc * L, L)]
                    acc_h[jc] = acc_h[jc] + hk * m
                    acc_t[jc] = acc_t[jc] + tk * m
            rvc = [rv_b[tri, pl.ds(jc * L, L)] for jc in range(jc_n)]
            grams = [jnp.zeros((L,), jnp.float32) for _ in range(6)]
            for jc in range(jc_n):
                ah, at, rv = acc_h[jc], acc_t[jc], rvc[jc]
                grams[0] = grams[0] + ah * ah
                grams[1] = grams[1] + at * at
                grams[2] = grams[2] + rv * rv
                grams[3] = grams[3] + ah * rv
                grams[4] = grams[4] + ah * at
                grams[5] = grams[5] + rv * at
            for q in range(6):
                red_b[q * pair + tri, :] = grams[q]
            return carry

        def epilogue(p, red_b, sign_vec):
            lane = lax.iota(jnp.int32, L)
            red = []
            for q in range(6):
                acc = jnp.zeros((L,), jnp.float32)
                for col in range(L):
                    acc = acc + plsc.load_gather(
                        red_b, [jnp.int32(q * pair) + lane,
                                jnp.full((L,), col, jnp.int32)])
                red.append(acc)
            sh, st, sr, phr, pht, prt = red

            def inv_norm(ssq):
                n = ssq * _rsqrt_nr(jnp.maximum(ssq, 1e-30))
                return 1.0 / jnp.maximum(n, 1e-12)

            ih, it_ = inv_norm(sh), inv_norm(st)
            irs = sign_vec * inv_norm(sr)
            nsq = (ih * ih * sh + it_ * it_ * st + irs * irs * sr
                   + 2.0 * (ih * irs * phr - ih * it_ * pht - irs * it_ * prt))
            nsq = jnp.maximum(nsq, 0.0)
            dist = nsq * _rsqrt_nr(jnp.maximum(nsq, 1e-30))
            score_all[pl.ds(p * pair, pair)] = gamma - dist

        def quarter_compute(j, ridxv_ref_next, he_b, te_b, rv_b, red_b,
                            prefetch_ok):
            slot = j % 4
            pre_slot = (j + 2) % 4
            ridx_ref, q_in_pair, guarded = ridxv_ref_next
            if guarded:
                @pl.when(prefetch_ok)
                def _():
                    issue_quarter(ridx_ref, q_in_pair, pre_slot)
            else:
                issue_quarter(ridx_ref, q_in_pair, pre_slot)
            wait_quarter(slot)

            @plsc.parallel_loop((j % 4) * qrows, (j % 4) * qrows + qrows)
            def _(t):
                triple_body(t, mslots[slot], t - (j % 4) * qrows,
                            he_b, te_b, rv_b, red_b, 0)

        def body(k, sign0):
            p0 = 2 * k
            p1 = 2 * k + 1
            not_last = k < n_bodies - 1
            ridx1, sign1 = pair_idx_sign(p1)
            store_quarter_idx(ridxv1, ridx1)
            issue_rows(p1, he1, te1, rv1, ridx1, semR1)
            wait_rows(semR0)
            quarter_compute(0, (ridxv0, 2, False), he0, te0, rv0, red0, True)
            quarter_compute(1, (ridxv0, 3, False), he0, te0, rv0, red0, True)
            quarter_compute(2, (ridxv1, 0, False), he0, te0, rv0, red0, True)
            quarter_compute(3, (ridxv1, 1, False), he0, te0, rv0, red0, True)
            epilogue(p0, red0, sign0)
            ridx2, sign2 = pair_idx_sign(p0 + 2)
            @pl.when(not_last)
            def _():
                store_quarter_idx(ridxv0, ridx2)
                issue_rows(p0 + 2, he0, te0, rv0, ridx2, semR0)
            wait_rows(semR1)
            quarter_compute(4, (ridxv1, 2, False), he1, te1, rv1, red1, True)
            quarter_compute(5, (ridxv1, 3, False), he1, te1, rv1, red1, True)
            quarter_compute(6, (ridxv0, 0, True), he1, te1, rv1, red1, not_last)
            quarter_compute(7, (ridxv0, 1, True), he1, te1, rv1, red1, not_last)
            epilogue(p1, red1, sign1)
            return sign2

        cp_h = pltpu.async_copy(h_hbm.at[pl.ds(base_w, tpw)], h_vm, sem_i)
        cp_t = pltpu.async_copy(t_hbm.at[pl.ds(base_w, tpw)], t_vm, sem_i)
        cp_r = pltpu.async_copy(r_hbm.at[pl.ds(base_w, tpw)], r_vm, sem_i)
        cp_h.wait()
        cp_t.wait()
        cp_r.wait()
        ridx0, sign0 = pair_idx_sign(0)
        store_quarter_idx(ridxv0, ridx0)
        issue_rows(0, he0, te0, rv0, ridx0, semR0)
        issue_quarter(ridxv0, 0, 0)
        issue_quarter(ridxv0, 1, 1)
        lax.fori_loop(0, n_bodies, body, sign0)
        pltpu.sync_copy(score_all, out_hbm.at[pl.ds(base_w, tpw)])

    return trans_r


_trans_r_full = None


def kernel(ent_embed, rel_embed, mat_embed, h, r, t):
    global _trans_r_full
    if _trans_r_full is None:
        _trans_r_full = _build(N_ENT, N_REL, DIM, B, GAMMA)
    return _trans_r_full(ent_embed, rel_embed, mat_embed, h, r, t)


# speedup vs baseline: 1.1909x; 1.0032x over previous
import functools

import jax
import jax.numpy as jnp
from jax import lax
from jax.experimental import pallas as pl
from jax.experimental.pallas import tpu as pltpu
from jax.experimental.pallas import tpu_sc as plsc

N_ENT = 1000000
N_REL = 1000
DIM = 64
GAMMA = 12.0
B = 16384

NC = 2
NS = 16
L = 16
NW = NC * NS


def _rsqrt_nr(x):
    i = plsc.bitcast(x, jnp.int32)
    i = jnp.int32(0x5F3759DF) - (i >> 1)
    y = plsc.bitcast(i, jnp.float32)
    for _ in range(3):
        y = y * (1.5 - 0.5 * x * y * y)
    return y


def _build(n_ent, n_rel, dim, b, gamma, interpret=False):
    tpw = b // NW
    pair = 16
    qrows = 4
    n_pairs = tpw // pair
    n_bodies = n_pairs // 2
    jc_n = dim // L
    row_bytes_rows = (3 * pair, dim)

    mesh = plsc.VectorSubcoreMesh(core_axis_name="c", subcore_axis_name="s")

    @functools.partial(
        pl.kernel,
        out_type=jax.ShapeDtypeStruct((b,), jnp.float32),
        mesh=mesh,
        interpret=interpret,
        compiler_params=pltpu.CompilerParams(
            needs_layout_passes=False,
            disable_bounds_checks=True,
            disable_semaphore_checks=True,
            skip_device_barrier=True),
        scratch_types=[
            pltpu.VMEM((tpw,), jnp.int32),
            pltpu.VMEM((tpw,), jnp.int32),
            pltpu.VMEM((tpw,), jnp.int32),
            pltpu.VMEM((2 * pair,), jnp.int32),
            pltpu.VMEM((2 * pair,), jnp.int32),
            pltpu.VMEM((pair, dim), jnp.float32),
            pltpu.VMEM((pair, dim), jnp.float32),
            pltpu.VMEM((pair, dim), jnp.float32),
            pltpu.VMEM((pair, dim), jnp.float32),
            pltpu.VMEM((pair, dim), jnp.float32),
            pltpu.VMEM((pair, dim), jnp.float32),
            pltpu.VMEM((qrows, dim * dim), jnp.float32),
            pltpu.VMEM((qrows, dim * dim), jnp.float32),
            pltpu.VMEM((qrows, dim * dim), jnp.float32),
            pltpu.VMEM((qrows, dim * dim), jnp.float32),
            pltpu.VMEM((6 * pair, L), jnp.float32),
            pltpu.VMEM((6 * pair, L), jnp.float32),
            pltpu.VMEM((tpw,), jnp.float32),
            pltpu.VMEM(row_bytes_rows, jnp.float32),
            pltpu.SemaphoreType.DMA,
            pltpu.SemaphoreType.DMA,
            pltpu.SemaphoreType.DMA,
            pltpu.SemaphoreType.DMA,
            pltpu.SemaphoreType.DMA,
            pltpu.SemaphoreType.DMA,
            pltpu.SemaphoreType.DMA,
        ],
    )
    def trans_r(ent_hbm, rel_hbm, mat_hbm, h_hbm, r_hbm, t_hbm, out_hbm,
                h_vm, t_vm, r_vm, ridxv0, ridxv1, he0, te0, rv0, he1, te1,
                rv1, m0, m1, m2, m3, red0, red1, score_all, drain_rows,
                sem_i, semR0, semR1, semM0, semM1, semM2, semM3):
        wid = lax.axis_index("s") * NC + lax.axis_index("c")
        base_w = wid * tpw
        mslots = [m0, m1, m2, m3]
        msems = [semM0, semM1, semM2, semM3]

        def pair_idx_sign(p):
            goff = jnp.minimum(p * pair, tpw - pair)
            rr = r_vm[pl.ds(goff, pair)]
            return lax.rem(rr, n_rel), jnp.where(
                rr < n_rel, 1.0, -1.0).astype(jnp.float32)

        def issue_rows(p, he_b, te_b, rv_b, ridx, semR):
            goff = p * pair
            hh = h_vm[pl.ds(goff, pair)]
            tt = t_vm[pl.ds(goff, pair)]
            for c in range(pair):
                pltpu.async_copy(ent_hbm.at[hh[c]], he_b.at[c], semR)
                pltpu.async_copy(ent_hbm.at[tt[c]], te_b.at[c], semR)
                pltpu.async_copy(rel_hbm.at[ridx[c]], rv_b.at[c], semR)

        def wait_rows(semR):
            pltpu.make_async_copy(
                ent_hbm.at[pl.ds(0, 3 * pair)], drain_rows, semR).wait()

        def store_quarter_idx(ridxv_ref, ridx):
            lane = lax.iota(jnp.int32, L)
            pos = 8 * (lane >> 2) + (lane & 3)
            plsc.store_scatter(ridxv_ref, [pos], ridx)

        def issue_quarter(ridxv_ref, q_in_pair, slot):
            pltpu.async_copy(
                mat_hbm.at[ridxv_ref.at[pl.ds(q_in_pair * 8, qrows)]],
                mslots[slot], msems[slot])

        def wait_quarter(slot):
            pltpu.make_async_copy(
                mat_hbm.at[pl.ds(0, qrows)], mslots[slot], msems[slot]).wait()

        def triple_body(tri, mat_ref, mi, he_b, te_b, rv_b, red_b, carry):
            hrow = [he_b[tri, pl.ds(c * L, L)] for c in range(jc_n)]
            trow = [te_b[tri, pl.ds(c * L, L)] for c in range(jc_n)]
            acc_h = [jnp.zeros((L,), jnp.float32) for _ in range(jc_n)]
            acc_t = [jnp.zeros((L,), jnp.float32) for _ in range(jc_n)]
            for k in range(dim):
                hk = hrow[k // L][k % L]
                tk = trow[k // L][k % L]
                for jc in range(jc_n):
                    m = mat_ref[mi, pl.ds(k * dim + jc * L, L)]
                    acc_h[jc] = acc_h[jc] + hk * m
                    acc_t[jc] = acc_t[jc] + tk * m
            rvc = [rv_b[tri, pl.ds(jc * L, L)] for jc in range(jc_n)]
            grams = [jnp.zeros((L,), jnp.float32) for _ in range(6)]
            for jc in range(jc_n):
                ah, at, rv = acc_h[jc], acc_t[jc], rvc[jc]
                grams[0] = grams[0] + ah * ah
                grams[1] = grams[1] + at * at
                grams[2] = grams[2] + rv * rv
                grams[3] = grams[3] + ah * rv
                grams[4] = grams[4] + ah * at
                grams[5] = grams[5] + rv * at
            for q in range(6):
                red_b[q * pair + tri, :] = grams[q]
            return carry

        def epilogue(p, red_b, sign_vec):
            lane = lax.iota(jnp.int32, L)
            red = []
            for q in range(6):
                acc = jnp.zeros((L,), jnp.float32)
                for col in range(L):
                    acc = acc + plsc.load_gather(
                        red_b, [jnp.int32(q * pair) + lane,
                                jnp.full((L,), col, jnp.int32)])
                red.append(acc)
            sh, st, sr, phr, pht, prt = red

            def inv_norm(ssq):
                n = ssq * _rsqrt_nr(jnp.maximum(ssq, 1e-30))
                return 1.0 / jnp.maximum(n, 1e-12)

            ih, it_ = inv_norm(sh), inv_norm(st)
            irs = sign_vec * inv_norm(sr)
            nsq = (ih * ih * sh + it_ * it_ * st + irs * irs * sr
                   + 2.0 * (ih * irs * phr - ih * it_ * pht - irs * it_ * prt))
            nsq = jnp.maximum(nsq, 0.0)
            dist = nsq * _rsqrt_nr(jnp.maximum(nsq, 1e-30))
            score_all[pl.ds(p * pair, pair)] = gamma - dist

        def quarter_compute(j, ridxv_ref_next, he_b, te_b, rv_b, red_b,
                            prefetch_ok):
            slot = j % 4
            pre_slot = (j + 2) % 4
            ridx_ref, q_in_pair, guarded = ridxv_ref_next
            if guarded:
                @pl.when(prefetch_ok)
                def _():
                    issue_quarter(ridx_ref, q_in_pair, pre_slot)
            else:
                issue_quarter(ridx_ref, q_in_pair, pre_slot)
            wait_quarter(slot)

            @plsc.parallel_loop((j % 4) * qrows, (j % 4) * qrows + qrows)
            def _(t):
                triple_body(t, mslots[slot], t - (j % 4) * qrows,
                            he_b, te_b, rv_b, red_b, 0)

        def body(k, sign0):
            p0 = 2 * k
            p1 = 2 * k + 1
            not_last = k < n_bodies - 1
            ridx1, sign1 = pair_idx_sign(p1)
            store_quarter_idx(ridxv1, ridx1)
            issue_rows(p1, he1, te1, rv1, ridx1, semR1)
            wait_rows(semR0)
            quarter_compute(0, (ridxv0, 2, False), he0, te0, rv0, red0, True)
            quarter_compute(1, (ridxv0, 3, False), he0, te0, rv0, red0, True)
            quarter_compute(2, (ridxv1, 0, False), he0, te0, rv0, red0, True)
            quarter_compute(3, (ridxv1, 1, False), he0, te0, rv0, red0, True)
            epilogue(p0, red0, sign0)
            ridx2, sign2 = pair_idx_sign(p0 + 2)
            @pl.when(not_last)
            def _():
                store_quarter_idx(ridxv0, ridx2)
                issue_rows(p0 + 2, he0, te0, rv0, ridx2, semR0)
            wait_rows(semR1)
            quarter_compute(4, (ridxv1, 2, False), he1, te1, rv1, red1, True)
            quarter_compute(5, (ridxv1, 3, False), he1, te1, rv1, red1, True)
            quarter_compute(6, (ridxv0, 0, True), he1, te1, rv1, red1, not_last)
            quarter_compute(7, (ridxv0, 1, True), he1, te1, rv1, red1, not_last)
            epilogue(p1, red1, sign1)
            return sign2

        cp_h = pltpu.async_copy(h_hbm.at[pl.ds(base_w, tpw)], h_vm, sem_i)
        cp_t = pltpu.async_copy(t_hbm.at[pl.ds(base_w, tpw)], t_vm, sem_i)
        cp_r = pltpu.async_copy(r_hbm.at[pl.ds(base_w, tpw)], r_vm, sem_i)
        cp_h.wait()
        cp_t.wait()
        cp_r.wait()
        ridx0, sign0 = pair_idx_sign(0)
        store_quarter_idx(ridxv0, ridx0)
        issue_rows(0, he0, te0, rv0, ridx0, semR0)
        issue_quarter(ridxv0, 0, 0)
        issue_quarter(ridxv0, 1, 1)
        lax.fori_loop(0, n_bodies, body, sign0)
        pltpu.sync_copy(score_all, out_hbm.at[pl.ds(base_w, tpw)])

    return trans_r


_trans_r_full = None


def kernel(ent_embed, rel_embed, mat_embed, h, r, t):
    global _trans_r_full
    if _trans_r_full is None:
        _trans_r_full = _build(N_ENT, N_REL, DIM, B, GAMMA)
    return _trans_r_full(ent_embed, rel_embed, mat_embed, h, r, t)
